# baseline (speedup 1.0000x reference)
import jax
import jax.numpy as jnp
from jax import lax
from jax.experimental import pallas as pl
from jax.experimental.pallas import tpu as pltpu
from jax.experimental.pallas import tpu_sc as plsc

DIM_ = 256
K_ = 8192
ROWS_PER_BLOCK = 256
GATHER_WINDOW = 128


def _e2h_body(e_ref, o_ref):
    e = e_ref[...]
    o_ref[...] = 0.5 * jnp.sum(e * e, axis=0, keepdims=True)


def _e2h(embed):
    return pl.pallas_call(
        _e2h_body,
        out_shape=jax.ShapeDtypeStruct((1, K_), jnp.float32),
    )(embed)


def _tc_body(x_ref, e_ref, e2h_ref, idx_ref, diff_ref):
    i = pl.program_id(0)
    x = x_ref[...]
    e = e_ref[...]
    prod = lax.dot_general(
        x, e, (((1,), (0,)), ((), ())),
        preferred_element_type=jnp.float32,
        precision=lax.Precision.DEFAULT,
    )
    s = prod - e2h_ref[...]
    idx_ref[0, 0, :] = jnp.argmax(s, axis=1).astype(jnp.int32)
    maxs = jnp.max(s, axis=1)
    f2 = jnp.sum(x * x, axis=1)
    part = (jnp.sum(f2) - 2.0 * jnp.sum(maxs)).reshape(1, 1)

    @pl.when(i == 0)
    def _():
        diff_ref[...] = part

    @pl.when(i != 0)
    def _():
        diff_ref[...] += part


def _tc_assign(x, embed, e2h):
    n = x.shape[0]
    nblocks = n // ROWS_PER_BLOCK
    return pl.pallas_call(
        _tc_body,
        grid=(nblocks,),
        in_specs=[
            pl.BlockSpec((ROWS_PER_BLOCK, DIM_), lambda i: (i, 0)),
            pl.BlockSpec((DIM_, K_), lambda i: (0, 0)),
            pl.BlockSpec((1, K_), lambda i: (0, 0)),
        ],
        out_specs=[
            pl.BlockSpec((1, 1, ROWS_PER_BLOCK), lambda i: (i, 0, 0)),
            pl.BlockSpec((1, 1), lambda i: (0, 0)),
        ],
        out_shape=[
            jax.ShapeDtypeStruct((nblocks, 1, ROWS_PER_BLOCK), jnp.int32),
            jax.ShapeDtypeStruct((1, 1), jnp.float32),
        ],
    )(x, embed, e2h)


def _sc_gather(table, idx_row):
    n = idx_row.shape[1]
    mesh = plsc.VectorSubcoreMesh(core_axis_name="core",
                                  subcore_axis_name="subcore")

    @pl.kernel(
        out_type=jax.ShapeDtypeStruct((n, DIM_), jnp.float32),
        mesh=mesh,
    )
    def k(table_hbm, i_hbm, o_hbm):
        def body(i_vmem, o_vmem):
            pltpu.sync_copy(table_hbm.at[i_vmem.at[0]], o_vmem)

        pltpu.emit_pipeline(
            body,
            grid=(n // GATHER_WINDOW,),
            in_specs=[pl.BlockSpec((1, GATHER_WINDOW),
                                   index_map=lambda i: (0, i))],
            out_specs=[pl.BlockSpec((GATHER_WINDOW, DIM_),
                                    index_map=lambda i: (i, 0))],
            core_axis_name=("core", "subcore"),
            dimension_semantics=(pltpu.PARALLEL,),
        )(i_hbm, o_hbm)

    return k(table, idx_row)


def kernel(input, embed):
    x = input.reshape(-1, DIM_)
    n = x.shape[0]
    nchunks = 4
    cn = n // nchunks
    e2h = _e2h(embed)
    embed_t = embed.T
    idx_parts, q_parts, diff_parts = [], [], []
    for c in range(nchunks):
        idx3, dsum = _tc_assign(x[c * cn:(c + 1) * cn], embed, e2h)
        idx_flat = idx3.reshape(-1)
        idx_parts.append(idx_flat)
        q_parts.append(_sc_gather(embed_t, idx_flat.reshape(1, cn)))
        diff_parts.append(dsum)
    embed_ind = jnp.concatenate(idx_parts).reshape(input.shape[:-1])
    quantize_st = jnp.concatenate(q_parts).reshape(input.shape)
    diff = sum(d[0, 0] for d in diff_parts) / jnp.float32(n * DIM_)
    return (quantize_st, diff, embed_ind, -1)


# speedup vs baseline: 1.1010x; 1.1010x over previous
import jax
import jax.numpy as jnp
from jax import lax
from jax.experimental import pallas as pl
from jax.experimental.pallas import tpu as pltpu
from jax.experimental.pallas import tpu_sc as plsc

DIM_ = 256
K_ = 8192
ROWS_PER_BLOCK = 512
GATHER_WINDOW = 128


def _e2h_body(e_ref, o_ref):
    e = e_ref[...]
    o_ref[...] = 0.5 * jnp.sum(e * e, axis=0, keepdims=True)


def _e2h(embed):
    return pl.pallas_call(
        _e2h_body,
        out_shape=jax.ShapeDtypeStruct((1, K_), jnp.float32),
    )(embed)


def _tc_body(x_ref, e_ref, e2h_ref, idx_ref, diff_ref):
    i = pl.program_id(0)
    x = x_ref[...]
    e = e_ref[...]
    prod = lax.dot_general(
        x, e, (((1,), (0,)), ((), ())),
        preferred_element_type=jnp.float32,
        precision=lax.Precision.DEFAULT,
    )
    s = prod - e2h_ref[...]
    idx_ref[0, 0, :] = jnp.argmax(s, axis=1).astype(jnp.int32)
    maxs = jnp.max(s, axis=1)
    f2 = jnp.sum(x * x, axis=1)
    part = (jnp.sum(f2) - 2.0 * jnp.sum(maxs)).reshape(1, 1)

    @pl.when(i == 0)
    def _():
        diff_ref[...] = part

    @pl.when(i != 0)
    def _():
        diff_ref[...] += part


def _tc_assign(x, embed, e2h):
    n = x.shape[0]
    nblocks = n // ROWS_PER_BLOCK
    return pl.pallas_call(
        _tc_body,
        grid=(nblocks,),
        in_specs=[
            pl.BlockSpec((ROWS_PER_BLOCK, DIM_), lambda i: (i, 0)),
            pl.BlockSpec((DIM_, K_), lambda i: (0, 0)),
            pl.BlockSpec((1, K_), lambda i: (0, 0)),
        ],
        out_specs=[
            pl.BlockSpec((1, 1, ROWS_PER_BLOCK), lambda i: (i, 0, 0)),
            pl.BlockSpec((1, 1), lambda i: (0, 0)),
        ],
        out_shape=[
            jax.ShapeDtypeStruct((nblocks, 1, ROWS_PER_BLOCK), jnp.int32),
            jax.ShapeDtypeStruct((1, 1), jnp.float32),
        ],
    )(x, embed, e2h)


def _sc_gather(table, idx_row):
    n = idx_row.shape[1]
    mesh = plsc.VectorSubcoreMesh(core_axis_name="core",
                                  subcore_axis_name="subcore")

    @pl.kernel(
        out_type=jax.ShapeDtypeStruct((n, DIM_), jnp.float32),
        mesh=mesh,
    )
    def k(table_hbm, i_hbm, o_hbm):
        def body(i_vmem, o_vmem):
            pltpu.sync_copy(table_hbm.at[i_vmem.at[0]], o_vmem)

        pltpu.emit_pipeline(
            body,
            grid=(n // GATHER_WINDOW,),
            in_specs=[pl.BlockSpec((1, GATHER_WINDOW),
                                   index_map=lambda i: (0, i))],
            out_specs=[pl.BlockSpec((GATHER_WINDOW, DIM_),
                                    index_map=lambda i: (i, 0))],
            core_axis_name=("core", "subcore"),
            dimension_semantics=(pltpu.PARALLEL,),
        )(i_hbm, o_hbm)

    return k(table, idx_row)


def kernel(input, embed):
    x = input.reshape(-1, DIM_)
    n = x.shape[0]
    e2h = _e2h(embed)
    idx3, diff_sum = _tc_assign(x, embed, e2h)
    idx_flat = idx3.reshape(-1)
    embed_ind = idx_flat.reshape(input.shape[:-1])
    quantize = _sc_gather(embed.T, idx_flat.reshape(1, n))
    quantize_st = quantize.reshape(input.shape)
    diff = diff_sum[0, 0] / jnp.float32(n * DIM_)
    return (quantize_st, diff, embed_ind, -1)
